# Initial kernel scaffold; baseline (speedup 1.0000x reference)
import jax
import jax.numpy as jnp
from jax.experimental import pallas as pl


def kernel(x, index, W, b):
    raise NotImplementedError("write your pallas kernel here")


# speedup vs baseline: 2.4951x; 2.4951x over previous
import functools

import jax
import jax.numpy as jnp
from jax import lax
from jax.experimental import pallas as pl
from jax.experimental.pallas import tpu as pltpu
from jax.experimental.pallas import tpu_sc as plsc

N_TOKENS = 8192
D_IN = 2048
D_OUT = 2048
N_EXP = 8
TILE = 256
P_ROWS = N_TOKENS + N_EXP * TILE
NT = P_ROWS // TILE
TE_LEN = 48
NC = 2
NS = 16
L = 16
NW = NC * NS
ROUTE_CHUNK = N_TOKENS // NS
MOVE_CHUNK = N_TOKENS // NW
ROWS_PER_DMA = 16
NDMA = MOVE_CHUNK // ROWS_PER_DMA

_mesh = plsc.VectorSubcoreMesh(
    core_axis_name="c", subcore_axis_name="s", num_cores=NC, num_subcores=NS)
_SC_PARAMS = pltpu.CompilerParams(needs_layout_passes=False)


def _eq_mask(v, e):
    return 1 - jnp.minimum(jnp.abs(v - e), 1)


@functools.partial(
    pl.kernel,
    out_type=(
        jax.ShapeDtypeStruct((N_TOKENS,), jnp.int32),
        jax.ShapeDtypeStruct((TE_LEN,), jnp.int32),
        jax.ShapeDtypeStruct((NS, L), jnp.int32),
    ),
    mesh=_mesh,
    compiler_params=_SC_PARAMS,
    scratch_types=[
        pltpu.VMEM((ROUTE_CHUNK,), jnp.int32),
        pltpu.VMEM((L,), jnp.int32),
        pltpu.VMEM((NS, L), jnp.int32),
        pltpu.VMEM((L,), jnp.int32),
        pltpu.VMEM((L,), jnp.int32),
        pltpu.VMEM((ROUTE_CHUNK,), jnp.int32),
        pltpu.VMEM((TE_LEN,), jnp.int32),
    ],
)
def _route(index_hbm, dest_hbm, te_hbm, cnt_hbm,
           idx_v, cnt_v, allcnt_v, gat_v, pe_v, dest_sc, te_v):
    cid = lax.axis_index("c")
    sid = lax.axis_index("s")
    work = cid == 0
    lane = lax.iota(jnp.int32, L)

    @pl.when(work)
    def _histogram():
        pltpu.sync_copy(index_hbm.at[pl.ds(sid * ROUTE_CHUNK, ROUTE_CHUNK)],
                        idx_v)

        def hist(j, counts):
            off = pl.multiple_of(j * L, L)
            v = idx_v[pl.ds(off, L)]
            for e in range(N_EXP):
                pc = jnp.sum(_eq_mask(v, e))
                counts = counts + _eq_mask(lane, e) * pc
            return counts

        counts = lax.fori_loop(0, ROUTE_CHUNK // L, hist,
                               jnp.zeros((L,), jnp.int32))
        cnt_v[...] = counts
        pltpu.sync_copy(cnt_v, cnt_hbm.at[sid])

    plsc.subcore_barrier()

    @pl.when(work)
    def _assign():
        pltpu.sync_copy(cnt_hbm, allcnt_v)
        tot = jnp.zeros((L,), jnp.int32)
        pre = jnp.zeros((L,), jnp.int32)
        for r in range(NS):
            row = allcnt_v[r]
            tot = tot + row
            before = jnp.minimum(jnp.maximum(sid - r, 0), 1)
            pre = pre + row * before
        keep = 1 - jnp.minimum(lane >> 3, 1)
        tot = tot * keep
        padded = ((tot + (TILE - 1)) >> 8) << 8
        pad_end = plsc.cumsum(padded)
        ps = pad_end - padded
        gat_v[...] = ps + pre
        pe_v[...] = ps

        def dest_row(j, _):
            off = pl.multiple_of(j * L, L)
            v = idx_v[pl.ds(off, L)]
            g = plsc.load_gather(gat_v, [v])
            gat = gat_v[...]
            rank = jnp.zeros((L,), jnp.int32)
            for e in range(N_EXP):
                m = _eq_mask(v, e)
                c = plsc.cumsum(m)
                rank = rank + m * (c - 1)
                gat = gat + _eq_mask(lane, e) * jnp.sum(m)
            gat_v[...] = gat
            dest_sc[pl.ds(off, L)] = g + rank
            return 0

        lax.fori_loop(0, ROUTE_CHUNK // L, dest_row, 0)
        pltpu.sync_copy(dest_sc,
                        dest_hbm.at[pl.ds(sid * ROUTE_CHUNK, ROUTE_CHUNK)])

        @pl.when(sid == 0)
        def _tile_experts():
            for j in range(TE_LEN // L):
                pos = (lax.iota(jnp.int32, L) + j * L) * TILE
                te = jnp.zeros((L,), jnp.int32)
                for e in range(N_EXP):
                    pe = plsc.load_gather(
                        pe_v, [jnp.full((L,), e + 1, jnp.int32)])
                    te = te + jnp.minimum(jnp.maximum(pos - pe + 1, 0), 1)
                te_v[pl.ds(j * L, L)] = jnp.minimum(te, N_EXP - 1)
            pltpu.sync_copy(te_v, te_hbm)


@functools.partial(
    pl.kernel,
    out_type=jax.ShapeDtypeStruct((P_ROWS, D_IN), jnp.float32),
    mesh=_mesh,
    compiler_params=_SC_PARAMS,
    scratch_types=[
        pltpu.VMEM((MOVE_CHUNK,), jnp.int32),
        pltpu.VMEM((ROWS_PER_DMA, D_IN), jnp.float32),
        pltpu.SemaphoreType.DMA,
    ],
)
def _scatter_rows(x_hbm, dest_hbm, xs_hbm, dest_v, rows_v, sem):
    cid = lax.axis_index("c")
    sid = lax.axis_index("s")
    wid = sid * NC + cid
    pltpu.sync_copy(dest_hbm.at[pl.ds(wid * MOVE_CHUNK, MOVE_CHUNK)], dest_v)
    for ch in range(NDMA):
        pltpu.sync_copy(
            x_hbm.at[pl.ds(wid * MOVE_CHUNK + ch * ROWS_PER_DMA,
                           ROWS_PER_DMA)],
            rows_v)
        idx = dest_v[pl.ds(ch * ROWS_PER_DMA, ROWS_PER_DMA)]
        pltpu.async_copy(rows_v, xs_hbm.at[idx], sem).wait()


@functools.partial(
    pl.kernel,
    out_type=jax.ShapeDtypeStruct((N_TOKENS, D_OUT), jnp.float32),
    mesh=_mesh,
    compiler_params=_SC_PARAMS,
    scratch_types=[
        pltpu.VMEM((MOVE_CHUNK,), jnp.int32),
        pltpu.VMEM((ROWS_PER_DMA, D_OUT), jnp.float32),
        pltpu.SemaphoreType.DMA,
    ],
)
def _gather_rows(ys_hbm, dest_hbm, y_hbm, dest_v, rows_v, sem):
    cid = lax.axis_index("c")
    sid = lax.axis_index("s")
    wid = sid * NC + cid
    pltpu.sync_copy(dest_hbm.at[pl.ds(wid * MOVE_CHUNK, MOVE_CHUNK)], dest_v)
    for ch in range(NDMA):
        idx = dest_v[pl.ds(ch * ROWS_PER_DMA, ROWS_PER_DMA)]
        pltpu.async_copy(ys_hbm.at[idx], rows_v, sem).wait()
        pltpu.sync_copy(
            rows_v,
            y_hbm.at[pl.ds(wid * MOVE_CHUNK + ch * ROWS_PER_DMA,
                           ROWS_PER_DMA)])


BN = 1024


def _mm_body(te_ref, x_ref, w_ref, b_ref, o_ref):
    acc = lax.dot_general(x_ref[...], w_ref[0],
                          (((1,), (1,)), ((), ())),
                          preferred_element_type=jnp.float32)
    o_ref[...] = acc + b_ref[0]


def _grouped_matmul(te, xs, W, b):
    return pl.pallas_call(
        _mm_body,
        grid_spec=pltpu.PrefetchScalarGridSpec(
            num_scalar_prefetch=1,
            grid=(D_OUT // BN, NT),
            in_specs=[
                pl.BlockSpec((TILE, D_IN), lambda n, r, te_ref: (r, 0)),
                pl.BlockSpec((1, BN, D_IN),
                             lambda n, r, te_ref: (te_ref[r], n, 0)),
                pl.BlockSpec((1, 1, BN),
                             lambda n, r, te_ref: (te_ref[r], 0, n)),
            ],
            out_specs=pl.BlockSpec((TILE, BN), lambda n, r, te_ref: (r, n)),
        ),
        out_shape=jax.ShapeDtypeStruct((P_ROWS, D_OUT), jnp.float32),
    )(te, xs, W, b.reshape(N_EXP, 1, D_OUT))


@jax.jit
def kernel(x, index, W, b):
    dest, te, _ = _route(index)
    xs = _scatter_rows(x, dest)
    ys = _grouped_matmul(te, xs, W, b)
    return _gather_rows(ys, dest)
